# baseline (speedup 1.0000x reference)
import functools

import jax
import jax.numpy as jnp
from jax import lax
from jax.experimental import pallas as pl
from jax.experimental.pallas import tpu as pltpu
from jax.experimental.pallas import tpu_sc as plsc

IN_DIM = 2048
HID = 1024
E = 8
NC = 2
GATE_H = 256
TOKENS = 4096
ASSIGN = 2 * TOKENS
BLK = 256
NBLK = ASSIGN // BLK + E
CAP = NBLK * BLK
PAD = CAP - ASSIGN
NW = 32


def _gelu(v):
    return v * 0.5 * (1.0 + lax.erf(v * 0.7071067811865476))


def _route_kernel(x_ref, Wg1_ref, bg1_ref, Wg2_ref, bg2_ref,
                  w_ref, pos_ref, pad_ref, be_ref):
    x = x_ref[...]
    g = _gelu(jnp.dot(x, Wg1_ref[...], preferred_element_type=jnp.float32)
              + bg1_ref[...])
    gl = jnp.dot(g, Wg2_ref[...], preferred_element_type=jnp.float32) \
        + bg2_ref[...]

    iota_e = lax.broadcasted_iota(jnp.int32, gl.shape, 1)
    m1 = jnp.max(gl, axis=-1, keepdims=True)
    i1 = jnp.min(jnp.where(gl == m1, iota_e, E), axis=-1, keepdims=True)
    oh1 = (iota_e == i1)
    masked = jnp.where(oh1, -jnp.inf, gl)
    m2 = jnp.max(masked, axis=-1, keepdims=True)
    i2 = jnp.min(jnp.where(masked == m2, iota_e, E), axis=-1, keepdims=True)
    oh2 = (iota_e == i2)
    e2 = jnp.exp(m2 - m1)
    w1 = 1.0 / (1.0 + e2)
    w2 = e2 * w1
    w_ref[...] = jnp.concatenate([w1, w2], axis=1)

    o1 = oh1.astype(jnp.float32)
    o2 = oh2.astype(jnp.float32)
    osum = o1 + o2
    inc = osum
    s = 1
    while s < TOKENS:
        inc = inc + jnp.concatenate(
            [jnp.zeros((s, E), jnp.float32), inc[:-s, :]], axis=0)
        s *= 2
    excl = inc - osum
    counts = inc[TOKENS - 1:TOKENS, :]

    padded = jnp.floor((counts + (BLK - 1)) / BLK) * BLK
    tri_e = (lax.broadcasted_iota(jnp.int32, (E, E), 0)
             < lax.broadcasted_iota(jnp.int32, (E, E), 1)).astype(jnp.float32)
    off = jnp.dot(padded, tri_e, preferred_element_type=jnp.float32)
    end = off + padded

    base = off + excl
    pos0 = jnp.sum(jnp.where(oh1, base, 0.0), axis=1, keepdims=True)
    pos1 = jnp.sum(jnp.where(oh2, base + o1, 0.0), axis=1, keepdims=True)
    pos_ref[...] = jnp.concatenate([pos0, pos1], axis=1).astype(jnp.int32)

    total = jnp.sum(padded, axis=1, keepdims=True)
    gsz = jnp.concatenate([padded - counts, CAP - total], axis=1)
    gstart = jnp.concatenate([off + counts, total], axis=1)
    tri_g = (lax.broadcasted_iota(jnp.int32, (E + 1, E + 1), 0)
             < lax.broadcasted_iota(jnp.int32, (E + 1, E + 1), 1)
             ).astype(jnp.float32)
    cumg = jnp.dot(gsz, tri_g, preferred_element_type=jnp.float32)
    i_pad = lax.broadcasted_iota(jnp.int32, (PAD, 1), 0).astype(jnp.float32)
    in_gap = jnp.logical_and(cumg <= i_pad, i_pad < cumg + gsz)
    pad_pos = jnp.sum(jnp.where(in_gap, gstart - cumg, 0.0), axis=1) \
        + i_pad[:, 0]
    pad_ref[...] = pad_pos.astype(jnp.int32)[None, :]

    jb = lax.broadcasted_iota(jnp.int32, (NBLK, 1), 0).astype(jnp.float32) * BLK
    be = jnp.sum((end <= jb).astype(jnp.int32), axis=1)
    be_ref[...] = jnp.minimum(be, E - 1)[None, :]


def _route(x, Wg1, bg1, Wg2, bg2):
    return pl.pallas_call(
        _route_kernel,
        out_shape=(
            jax.ShapeDtypeStruct((TOKENS, 2), jnp.float32),
            jax.ShapeDtypeStruct((TOKENS, 2), jnp.int32),
            jax.ShapeDtypeStruct((1, PAD), jnp.int32),
            jax.ShapeDtypeStruct((1, NBLK), jnp.int32),
        ),
    )(x, Wg1, bg1.reshape(1, GATE_H), Wg2, bg2.reshape(1, E))


_SC_MESH = dict(core_axis_name="c", subcore_axis_name="s")


def _sc_wid():
    return lax.axis_index("s") * 2 + lax.axis_index("c")


def _k2a_body(idx_hbm, out_hbm, iv0, vv0, iv1, vv1, iv2, vv2, sem):
    wid = _sc_wid()
    base = wid * (CAP // NW)
    iota16 = lax.iota(jnp.int32, 16)
    copies = []
    for ofs, n, iv, vv in ((0, 128, iv0, vv0),
                           (128, 128, iv1, vv1),
                           (256, 64, iv2, vv2)):
        pltpu.sync_copy(idx_hbm.at[pl.ds(base + ofs, n)], iv)
        for s in range(n // 16):
            a_vec = (base + ofs + s * 16) + iota16
            v = jnp.where(a_vec < ASSIGN,
                          lax.shift_right_logical(a_vec, 1), 0)
            vv[pl.ds(s * 16, 16)] = v
        copies.append(pltpu.async_copy(vv, out_hbm.at[iv], sem))
    for c in copies:
        c.wait()


def _sc_scatter(idx_all):
    k = functools.partial(
        pl.kernel,
        mesh=plsc.VectorSubcoreMesh(**_SC_MESH),
        out_type=jax.ShapeDtypeStruct((CAP,), jnp.int32),
        scratch_types=[
            pltpu.VMEM((128,), jnp.int32),
            pltpu.VMEM((128,), jnp.int32),
            pltpu.VMEM((128,), jnp.int32),
            pltpu.VMEM((128,), jnp.int32),
            pltpu.VMEM((64,), jnp.int32),
            pltpu.VMEM((64,), jnp.int32),
            pltpu.SemaphoreType.DMA,
        ],
    )(_k2a_body)
    return k(idx_all)


_ROWS_PER_W = CAP // NW
_GCHUNK = 40
_NCH = _ROWS_PER_W // _GCHUNK
_IN_P = IN_DIM // 2


def _k2b_body(x_hbm, tok_hbm, xs_hbm, tokv0, tokv1, rows0, rows1,
              gsem, wsem):
    wid = _sc_wid()
    base = wid * _ROWS_PER_W
    tokv = (tokv0, tokv1)
    rows = (rows0, rows1)
    pltpu.sync_copy(tok_hbm.at[pl.ds(base, _GCHUNK)], tokv0)
    gh = pltpu.async_copy(x_hbm.at[tokv0], rows0, gsem)
    wh_prev = None
    for c in range(_NCH):
        cur = c % 2
        nxt = (c + 1) % 2
        if c + 1 < _NCH:
            pltpu.sync_copy(
                tok_hbm.at[pl.ds(base + (c + 1) * _GCHUNK, _GCHUNK)],
                tokv[nxt])
        gh.wait()
        wh = pltpu.async_copy(
            rows[cur], xs_hbm.at[pl.ds(base + c * _GCHUNK, _GCHUNK)], wsem)
        if c + 1 < _NCH:
            if wh_prev is not None:
                wh_prev.wait()
            gh = pltpu.async_copy(x_hbm.at[tokv[nxt]], rows[nxt], gsem)
        else:
            if wh_prev is not None:
                wh_prev.wait()
        wh_prev = wh
    wh_prev.wait()


def _sc_gather(x_packed, sorted_tok):
    k = functools.partial(
        pl.kernel,
        mesh=plsc.VectorSubcoreMesh(**_SC_MESH),
        out_type=jax.ShapeDtypeStruct((CAP, _IN_P), jnp.int32),
        scratch_types=[
            pltpu.VMEM((_GCHUNK,), jnp.int32),
            pltpu.VMEM((_GCHUNK,), jnp.int32),
            pltpu.VMEM((_GCHUNK, _IN_P), jnp.int32),
            pltpu.VMEM((_GCHUNK, _IN_P), jnp.int32),
            pltpu.SemaphoreType.DMA,
            pltpu.SemaphoreType.DMA,
        ],
    )(_k2b_body)
    return k(x_packed, sorted_tok)


def _mlp_kernel(be_ref, xs_ref, W1_ref, b1_ref, W2_ref, b2_ref,
                W3_ref, b3_ref, out_ref):
    h1 = _gelu(jnp.dot(xs_ref[...].astype(jnp.float32), W1_ref[0],
                       preferred_element_type=jnp.float32) + b1_ref[0])
    h2 = _gelu(jnp.dot(h1, W2_ref[0],
                       preferred_element_type=jnp.float32) + b2_ref[0])
    out_ref[...] = jnp.dot(h2, W3_ref[0],
                           preferred_element_type=jnp.float32) + b3_ref[0]


def _expert_mlp(be, xs, W1, b1, W2, b2, W3, b3):
    grid_spec = pltpu.PrefetchScalarGridSpec(
        num_scalar_prefetch=1,
        grid=(NBLK,),
        in_specs=[
            pl.BlockSpec((BLK, IN_DIM), lambda j, be: (j, 0)),
            pl.BlockSpec((1, IN_DIM, HID), lambda j, be: (be[j], 0, 0)),
            pl.BlockSpec((1, 1, HID), lambda j, be: (be[j], 0, 0)),
            pl.BlockSpec((1, HID, HID // 2), lambda j, be: (be[j], 0, 0)),
            pl.BlockSpec((1, 1, HID // 2), lambda j, be: (be[j], 0, 0)),
            pl.BlockSpec((1, HID // 2, NC), lambda j, be: (be[j], 0, 0)),
            pl.BlockSpec((1, 1, NC), lambda j, be: (be[j], 0, 0)),
        ],
        out_specs=pl.BlockSpec((BLK, NC), lambda j, be: (j, 0)),
    )
    return pl.pallas_call(
        _mlp_kernel,
        grid_spec=grid_spec,
        out_shape=jax.ShapeDtypeStruct((CAP, NC), jnp.float32),
    )(be, xs, W1, b1.reshape(E, 1, HID), W2, b2.reshape(E, 1, HID // 2),
      W3, b3.reshape(E, 1, NC))


_TOK_PER_W = TOKENS // NW


def _k4_body(o3_hbm, w0_hbm, w1_hbm, p0_hbm, p1_hbm, t_hbm, out_hbm,
             w0v, w1v, p0v, p1v, idxb, v00, v01, v10, v11, tv, ob, sem):
    wid = _sc_wid()
    tb = wid * _TOK_PER_W
    pltpu.sync_copy(w0_hbm.at[pl.ds(tb, _TOK_PER_W)], w0v)
    pltpu.sync_copy(w1_hbm.at[pl.ds(tb, _TOK_PER_W)], w1v)
    pltpu.sync_copy(p0_hbm.at[pl.ds(tb, _TOK_PER_W)], p0v)
    pltpu.sync_copy(p1_hbm.at[pl.ds(tb, _TOK_PER_W)], p1v)
    pltpu.sync_copy(t_hbm, tv)
    inv_t = 1.0 / jnp.maximum(tv[...], 1e-6)
    for pv, dsts in ((p0v, (v00, v01)), (p1v, (v10, v11))):
        for c, dst in enumerate(dsts):
            for g in range(_TOK_PER_W // 16):
                sl = pl.ds(g * 16, 16)
                idxb[sl] = pv[sl] * NC + c
            pltpu.async_copy(o3_hbm.at[idxb], dst, sem).wait()
    for c, (a, b) in enumerate(((v00, v10), (v01, v11))):
        for g in range(_TOK_PER_W // 16):
            sl = pl.ds(g * 16, 16)
            ob[sl] = (w0v[sl] * a[sl] + w1v[sl] * b[sl]) * inv_t
        pltpu.sync_copy(ob, out_hbm.at[pl.ds(c * TOKENS + tb, _TOK_PER_W)])


def _sc_combine(o3_flat, w0, w1, p0, p1, temp16):
    k = functools.partial(
        pl.kernel,
        mesh=plsc.VectorSubcoreMesh(**_SC_MESH),
        out_type=jax.ShapeDtypeStruct((TOKENS * NC,), jnp.float32),
        scratch_types=[
            pltpu.VMEM((_TOK_PER_W,), jnp.float32),
            pltpu.VMEM((_TOK_PER_W,), jnp.float32),
            pltpu.VMEM((_TOK_PER_W,), jnp.int32),
            pltpu.VMEM((_TOK_PER_W,), jnp.int32),
            pltpu.VMEM((_TOK_PER_W,), jnp.int32),
            pltpu.VMEM((_TOK_PER_W,), jnp.float32),
            pltpu.VMEM((_TOK_PER_W,), jnp.float32),
            pltpu.VMEM((_TOK_PER_W,), jnp.float32),
            pltpu.VMEM((_TOK_PER_W,), jnp.float32),
            pltpu.VMEM((16,), jnp.float32),
            pltpu.VMEM((_TOK_PER_W,), jnp.float32),
            pltpu.SemaphoreType.DMA,
        ],
    )(_k4_body)
    return k(o3_flat, w0, w1, p0, p1, temp16)


def kernel(x, W1, b1, W2, b2, W3, b3, Wg1, bg1, Wg2, bg2, temperature):
    w, pos, pad, be = _route(x, Wg1, bg1, Wg2, bg2)
    idx_all = jnp.concatenate([pos.reshape(ASSIGN), pad.reshape(PAD)])
    sorted_tok = _sc_scatter(idx_all)
    x_packed = lax.bitcast_convert_type(
        x.astype(jnp.bfloat16).reshape(TOKENS, _IN_P, 2), jnp.int32)
    xs_packed = _sc_gather(x_packed, sorted_tok)
    xs = lax.bitcast_convert_type(xs_packed, jnp.bfloat16).reshape(CAP, IN_DIM)
    o3 = _expert_mlp(be.reshape(NBLK), xs, W1, b1, W2, b2, W3, b3)
    temp16 = jnp.broadcast_to(temperature.reshape(1), (16,))
    out = _sc_combine(o3.reshape(CAP * NC), w[:, 0], w[:, 1],
                      pos[:, 0], pos[:, 1], temp16)
    return out.reshape(NC, TOKENS).T


# speedup vs baseline: 2.3842x; 2.3842x over previous
import functools

import jax
import jax.numpy as jnp
from jax import lax
from jax.experimental import pallas as pl
from jax.experimental.pallas import tpu as pltpu
from jax.experimental.pallas import tpu_sc as plsc

IN_DIM = 2048
HID = 1024
E = 8
NC = 2
GATE_H = 256
TOKENS = 4096
ASSIGN = 2 * TOKENS
BLK = 256
NBLK = ASSIGN // BLK + E
CAP = NBLK * BLK
PAD = CAP - ASSIGN
NW = 32


def _gelu(v):
    return v * 0.5 * (1.0 + lax.erf(v * 0.7071067811865476))


_RB = 512
_RTB = TOKENS // _RB


def _route_a_kernel(x_ref, Wg1_ref, bg1_ref, Wg2_ref, bg2_ref,
                    w0_ref, w1_ref, e0_ref, e1_ref, sel0_ref, sel1_ref,
                    counts_ref, xpk_ref, carry_ref):
    tb = pl.program_id(0)
    x = x_ref[...]

    u = lax.bitcast_convert_type(x, jnp.int32)
    top_mask = jnp.int32(-65536)

    def _rbf(v):
        return (v + 0x7FFF + (lax.shift_right_logical(v, 16) & 1)) & top_mask

    xpk_ref[...] = lax.shift_right_logical(_rbf(u[:, :IN_DIM // 2]), 16) \
        | _rbf(u[:, IN_DIM // 2:])

    g = _gelu(jnp.dot(x, Wg1_ref[...], preferred_element_type=jnp.float32)
              + bg1_ref[...])
    gl = jnp.dot(g, Wg2_ref[...], preferred_element_type=jnp.float32) \
        + bg2_ref[...]

    iota_e = lax.broadcasted_iota(jnp.int32, gl.shape, 1)
    m1 = jnp.max(gl, axis=-1, keepdims=True)
    i1 = jnp.min(jnp.where(gl == m1, iota_e, E), axis=-1, keepdims=True)
    oh1 = (iota_e == i1)
    masked = jnp.where(oh1, -jnp.inf, gl)
    m2 = jnp.max(masked, axis=-1, keepdims=True)
    i2 = jnp.min(jnp.where(masked == m2, iota_e, E), axis=-1, keepdims=True)
    oh2 = (iota_e == i2)
    e2 = jnp.exp(m2 - m1)
    w1 = 1.0 / (1.0 + e2)
    w0_ref[...] = w1[:, 0]
    w1_ref[...] = (e2 * w1)[:, 0]
    iota_f = iota_e.astype(jnp.float32)
    e0_ref[...] = jnp.sum(jnp.where(oh1, iota_f, 0.0), axis=1)
    e1_ref[...] = jnp.sum(jnp.where(oh2, iota_f, 0.0), axis=1)

    osum = oh1.astype(jnp.float32) + oh2.astype(jnp.float32)
    inc = osum
    s = 1
    while s < _RB:
        inc = inc + jnp.concatenate(
            [jnp.zeros((s, E), jnp.float32), inc[:-s, :]], axis=0)
        s *= 2
    prev = jnp.where(tb == 0, jnp.zeros((1, E), jnp.float32), carry_ref[...])
    excl = (inc - osum) + prev
    sel0_ref[...] = jnp.sum(jnp.where(oh1, excl, 0.0), axis=1)
    sel1_ref[...] = jnp.sum(jnp.where(oh2, excl, 0.0), axis=1)
    new_carry = prev + inc[_RB - 1:_RB, :]
    carry_ref[...] = new_carry

    @pl.when(tb == _RTB - 1)
    def _():
        counts_ref[...] = new_carry


def _route_b_kernel(counts_ref, e0_ref, e1_ref, sel0_ref, sel1_ref,
                    p0_ref, p1_ref, pad_ref, be_ref):
    counts = counts_ref[...]
    padded = jnp.floor((counts + (BLK - 1)) / BLK) * BLK
    tri_e = (lax.broadcasted_iota(jnp.int32, (E, E), 0)
             < lax.broadcasted_iota(jnp.int32, (E, E), 1)).astype(jnp.float32)
    off = jnp.dot(padded, tri_e, preferred_element_type=jnp.float32)
    end = off + padded

    iota_e = lax.broadcasted_iota(jnp.int32, (TOKENS, E), 1).astype(jnp.float32)
    oh0 = (iota_e == e0_ref[...][:, None])
    oh1 = (iota_e == e1_ref[...][:, None])
    p0_ref[...] = (jnp.sum(jnp.where(oh0, off, 0.0), axis=1)
                   + sel0_ref[...]).astype(jnp.int32)
    p1_ref[...] = (jnp.sum(jnp.where(oh1, off, 0.0), axis=1)
                   + sel1_ref[...]).astype(jnp.int32)

    total = jnp.sum(padded, axis=1, keepdims=True)
    gsz = jnp.concatenate([padded - counts, CAP - total], axis=1)
    gstart = jnp.concatenate([off + counts, total], axis=1)
    tri_g = (lax.broadcasted_iota(jnp.int32, (E + 1, E + 1), 0)
             < lax.broadcasted_iota(jnp.int32, (E + 1, E + 1), 1)
             ).astype(jnp.float32)
    cumg = jnp.dot(gsz, tri_g, preferred_element_type=jnp.float32)
    i_pad = lax.broadcasted_iota(jnp.int32, (PAD, 1), 0).astype(jnp.float32)
    in_gap = jnp.logical_and(cumg <= i_pad, i_pad < cumg + gsz)
    pad_ref[...] = (jnp.sum(jnp.where(in_gap, gstart - cumg, 0.0), axis=1)
                    + i_pad[:, 0]).astype(jnp.int32)

    jb = lax.broadcasted_iota(jnp.int32, (NBLK, 1), 0).astype(jnp.float32) * BLK
    be = jnp.sum((end <= jb).astype(jnp.int32), axis=1)
    be_ref[...] = jnp.minimum(be, E - 1)


def _route(x, Wg1, bg1, Wg2, bg2):
    w0, w1, e0, e1, sel0, sel1, counts, xpk = pl.pallas_call(
        _route_a_kernel,
        grid=(_RTB,),
        in_specs=[
            pl.BlockSpec((_RB, IN_DIM), lambda tb: (tb, 0)),
            pl.BlockSpec((IN_DIM, GATE_H), lambda tb: (0, 0)),
            pl.BlockSpec((1, GATE_H), lambda tb: (0, 0)),
            pl.BlockSpec((GATE_H, E), lambda tb: (0, 0)),
            pl.BlockSpec((1, E), lambda tb: (0, 0)),
        ],
        out_specs=(
            pl.BlockSpec((_RB,), lambda tb: (tb,)),
            pl.BlockSpec((_RB,), lambda tb: (tb,)),
            pl.BlockSpec((_RB,), lambda tb: (tb,)),
            pl.BlockSpec((_RB,), lambda tb: (tb,)),
            pl.BlockSpec((_RB,), lambda tb: (tb,)),
            pl.BlockSpec((_RB,), lambda tb: (tb,)),
            pl.BlockSpec((1, E), lambda tb: (0, 0)),
            pl.BlockSpec((_RB, IN_DIM // 2), lambda tb: (tb, 0)),
        ),
        out_shape=(
            jax.ShapeDtypeStruct((TOKENS,), jnp.float32),
            jax.ShapeDtypeStruct((TOKENS,), jnp.float32),
            jax.ShapeDtypeStruct((TOKENS,), jnp.float32),
            jax.ShapeDtypeStruct((TOKENS,), jnp.float32),
            jax.ShapeDtypeStruct((TOKENS,), jnp.float32),
            jax.ShapeDtypeStruct((TOKENS,), jnp.float32),
            jax.ShapeDtypeStruct((1, E), jnp.float32),
            jax.ShapeDtypeStruct((TOKENS, IN_DIM // 2), jnp.int32),
        ),
        scratch_shapes=[pltpu.VMEM((1, E), jnp.float32)],
    )(x, Wg1, bg1.reshape(1, GATE_H), Wg2, bg2.reshape(1, E))
    p0, p1, pad, be = pl.pallas_call(
        _route_b_kernel,
        out_shape=(
            jax.ShapeDtypeStruct((TOKENS,), jnp.int32),
            jax.ShapeDtypeStruct((TOKENS,), jnp.int32),
            jax.ShapeDtypeStruct((PAD,), jnp.int32),
            jax.ShapeDtypeStruct((NBLK,), jnp.int32),
        ),
    )(counts, e0, e1, sel0, sel1)
    return w0, w1, p0, p1, pad, be, xpk


_SC_MESH = dict(core_axis_name="c", subcore_axis_name="s")


def _sc_wid():
    return lax.axis_index("s") * 2 + lax.axis_index("c")


def _k2a_body(idx_hbm, out_hbm, iv0, vv0, iv1, vv1, iv2, vv2, sem):
    wid = _sc_wid()
    base = wid * (CAP // NW)
    iota16 = lax.iota(jnp.int32, 16)
    copies = []
    for ofs, n, iv, vv in ((0, 128, iv0, vv0),
                           (128, 128, iv1, vv1),
                           (256, 64, iv2, vv2)):
        pltpu.sync_copy(idx_hbm.at[pl.ds(base + ofs, n)], iv)
        for s in range(n // 16):
            a_vec = (base + ofs + s * 16) + iota16
            v = jnp.where(a_vec < ASSIGN, a_vec & (TOKENS - 1), 0)
            vv[pl.ds(s * 16, 16)] = v
        copies.append(pltpu.async_copy(vv, out_hbm.at[iv], sem))
    for c in copies:
        c.wait()


def _sc_scatter(idx_all):
    k = functools.partial(
        pl.kernel,
        mesh=plsc.VectorSubcoreMesh(**_SC_MESH),
        out_type=jax.ShapeDtypeStruct((CAP,), jnp.int32),
        scratch_types=[
            pltpu.VMEM((128,), jnp.int32),
            pltpu.VMEM((128,), jnp.int32),
            pltpu.VMEM((128,), jnp.int32),
            pltpu.VMEM((128,), jnp.int32),
            pltpu.VMEM((64,), jnp.int32),
            pltpu.VMEM((64,), jnp.int32),
            pltpu.SemaphoreType.DMA,
        ],
    )(_k2a_body)
    return k(idx_all)


_ROWS_PER_W = CAP // NW
_GCHUNK = 40
_NCH = _ROWS_PER_W // _GCHUNK
_IN_P = IN_DIM // 2


def _k2b_body(x_hbm, tok_hbm, xs_hbm, tokv0, tokv1, rows0, rows1,
              gsem, wsem):
    wid = _sc_wid()
    base = wid * _ROWS_PER_W
    tokv = (tokv0, tokv1)
    rows = (rows0, rows1)
    pltpu.sync_copy(tok_hbm.at[pl.ds(base, _GCHUNK)], tokv0)
    gh = pltpu.async_copy(x_hbm.at[tokv0], rows0, gsem)
    wh_prev = None
    for c in range(_NCH):
        cur = c % 2
        nxt = (c + 1) % 2
        if c + 1 < _NCH:
            pltpu.sync_copy(
                tok_hbm.at[pl.ds(base + (c + 1) * _GCHUNK, _GCHUNK)],
                tokv[nxt])
        gh.wait()
        wh = pltpu.async_copy(
            rows[cur], xs_hbm.at[pl.ds(base + c * _GCHUNK, _GCHUNK)], wsem)
        if c + 1 < _NCH:
            if wh_prev is not None:
                wh_prev.wait()
            gh = pltpu.async_copy(x_hbm.at[tokv[nxt]], rows[nxt], gsem)
        else:
            if wh_prev is not None:
                wh_prev.wait()
        wh_prev = wh
    wh_prev.wait()


def _sc_gather(x_packed, sorted_tok):
    k = functools.partial(
        pl.kernel,
        mesh=plsc.VectorSubcoreMesh(**_SC_MESH),
        out_type=jax.ShapeDtypeStruct((CAP, _IN_P), jnp.int32),
        scratch_types=[
            pltpu.VMEM((_GCHUNK,), jnp.int32),
            pltpu.VMEM((_GCHUNK,), jnp.int32),
            pltpu.VMEM((_GCHUNK, _IN_P), jnp.int32),
            pltpu.VMEM((_GCHUNK, _IN_P), jnp.int32),
            pltpu.SemaphoreType.DMA,
            pltpu.SemaphoreType.DMA,
        ],
    )(_k2b_body)
    return k(x_packed, sorted_tok)


def _mlp_kernel(be_ref, xs_ref, W1_ref, b1_ref, W2_ref, b2_ref,
                W3_ref, b3_ref, o30_ref, o31_ref):
    u = xs_ref[...]
    lo = lax.bitcast_convert_type(lax.shift_left(u, 16), jnp.float32)
    hi = lax.bitcast_convert_type(u & jnp.int32(-65536), jnp.float32)
    xs = jnp.concatenate([lo, hi], axis=1)
    h1 = _gelu(jnp.dot(xs, W1_ref[0],
                       preferred_element_type=jnp.float32) + b1_ref[0])
    h2 = _gelu(jnp.dot(h1, W2_ref[0],
                       preferred_element_type=jnp.float32) + b2_ref[0])
    w3 = W3_ref[0]
    b3 = b3_ref[0]
    o30_ref[...] = jnp.sum(h2 * w3[:, 0][None, :], axis=1) + b3[0, 0]
    o31_ref[...] = jnp.sum(h2 * w3[:, 1][None, :], axis=1) + b3[0, 1]


def _expert_mlp(be, xs, W1, b1, W2, b2, W3, b3):
    grid_spec = pltpu.PrefetchScalarGridSpec(
        num_scalar_prefetch=1,
        grid=(NBLK,),
        in_specs=[
            pl.BlockSpec((BLK, IN_DIM // 2), lambda j, be: (j, 0)),
            pl.BlockSpec((1, IN_DIM, HID), lambda j, be: (be[j], 0, 0)),
            pl.BlockSpec((1, 1, HID), lambda j, be: (be[j], 0, 0)),
            pl.BlockSpec((1, HID, HID // 2), lambda j, be: (be[j], 0, 0)),
            pl.BlockSpec((1, 1, HID // 2), lambda j, be: (be[j], 0, 0)),
            pl.BlockSpec((1, HID // 2, NC), lambda j, be: (be[j], 0, 0)),
            pl.BlockSpec((1, 1, NC), lambda j, be: (be[j], 0, 0)),
        ],
        out_specs=(
            pl.BlockSpec((BLK,), lambda j, be: (j,)),
            pl.BlockSpec((BLK,), lambda j, be: (j,)),
        ),
    )
    return pl.pallas_call(
        _mlp_kernel,
        grid_spec=grid_spec,
        out_shape=(
            jax.ShapeDtypeStruct((CAP,), jnp.float32),
            jax.ShapeDtypeStruct((CAP,), jnp.float32),
        ),
    )(be, xs, W1, b1.reshape(E, 1, HID), W2, b2.reshape(E, 1, HID // 2),
      W3, b3.reshape(E, 1, NC))


_TOK_PER_W = TOKENS // NW


def _k4_body(o30_hbm, o31_hbm, w0_hbm, w1_hbm, p0_hbm, p1_hbm, t_hbm,
             out_hbm, w0v, w1v, p0v, p1v, v00, v01, v10, v11, tv, ob, sem):
    wid = _sc_wid()
    tb = wid * _TOK_PER_W
    pltpu.sync_copy(w0_hbm.at[pl.ds(tb, _TOK_PER_W)], w0v)
    pltpu.sync_copy(w1_hbm.at[pl.ds(tb, _TOK_PER_W)], w1v)
    pltpu.sync_copy(p0_hbm.at[pl.ds(tb, _TOK_PER_W)], p0v)
    pltpu.sync_copy(p1_hbm.at[pl.ds(tb, _TOK_PER_W)], p1v)
    pltpu.sync_copy(t_hbm, tv)
    inv_t = 1.0 / jnp.maximum(tv[...], 1e-6)
    copies = []
    for pv, plane, dst in ((p0v, o30_hbm, v00), (p0v, o31_hbm, v01),
                           (p1v, o30_hbm, v10), (p1v, o31_hbm, v11)):
        copies.append(pltpu.async_copy(plane.at[pv], dst, sem))
    for cp in copies:
        cp.wait()
    for c, (a, b) in enumerate(((v00, v10), (v01, v11))):
        for g in range(_TOK_PER_W // 16):
            sl = pl.ds(g * 16, 16)
            ob[sl] = (w0v[sl] * a[sl] + w1v[sl] * b[sl]) * inv_t
        pltpu.sync_copy(ob, out_hbm.at[pl.ds(c * TOKENS + tb, _TOK_PER_W)])


def _sc_combine(o30, o31, w0, w1, p0, p1, temp16):
    k = functools.partial(
        pl.kernel,
        mesh=plsc.VectorSubcoreMesh(**_SC_MESH),
        out_type=jax.ShapeDtypeStruct((TOKENS * NC,), jnp.float32),
        scratch_types=[
            pltpu.VMEM((_TOK_PER_W,), jnp.float32),
            pltpu.VMEM((_TOK_PER_W,), jnp.float32),
            pltpu.VMEM((_TOK_PER_W,), jnp.int32),
            pltpu.VMEM((_TOK_PER_W,), jnp.int32),
            pltpu.VMEM((_TOK_PER_W,), jnp.float32),
            pltpu.VMEM((_TOK_PER_W,), jnp.float32),
            pltpu.VMEM((_TOK_PER_W,), jnp.float32),
            pltpu.VMEM((_TOK_PER_W,), jnp.float32),
            pltpu.VMEM((16,), jnp.float32),
            pltpu.VMEM((_TOK_PER_W,), jnp.float32),
            pltpu.SemaphoreType.DMA,
        ],
    )(_k4_body)
    return k(o30, o31, w0, w1, p0, p1, temp16)


def kernel(x, W1, b1, W2, b2, W3, b3, Wg1, bg1, Wg2, bg2, temperature):
    w0, w1, p0, p1, pad, be, x_packed = _route(x, Wg1, bg1, Wg2, bg2)
    idx_all = jnp.concatenate([p0, p1, pad])
    sorted_tok = _sc_scatter(idx_all)
    xs_packed = _sc_gather(x_packed, sorted_tok)
    o30, o31 = _expert_mlp(be, xs_packed, W1, b1, W2, b2, W3, b3)
    temp16 = jnp.broadcast_to(temperature.reshape(1), (16,))
    out = _sc_combine(o30, o31, w0, w1, p0, p1, temp16)
    return out.reshape(NC, TOKENS).T


# speedup vs baseline: 4.4944x; 1.8851x over previous
import functools

import jax
import jax.numpy as jnp
from jax import lax
from jax.experimental import pallas as pl
from jax.experimental.pallas import tpu as pltpu
from jax.experimental.pallas import tpu_sc as plsc

IN_DIM = 2048
HID = 1024
E = 8
NC = 2
GATE_H = 256
TOKENS = 4096
ASSIGN = 2 * TOKENS
BLK = 256
NBLK = ASSIGN // BLK + E
CAP = NBLK * BLK
PAD = CAP - ASSIGN
NW = 32


def _gelu(v):
    return v * 0.5 * (1.0 + lax.erf(v * 0.7071067811865476))


_RB = 512
_RTB = TOKENS // _RB


def _route_a_kernel(x_ref, Wg1_ref, bg1_ref, Wg2_ref, bg2_ref,
                    w0_ref, w1_ref, e0_ref, e1_ref, sel0_ref, sel1_ref,
                    counts_ref, xpk_ref, carry_ref):
    tb = pl.program_id(0)
    x = x_ref[...]

    u = lax.bitcast_convert_type(x, jnp.int32)
    top_mask = jnp.int32(-65536)

    def _rbf(v):
        return (v + 0x7FFF + (lax.shift_right_logical(v, 16) & 1)) & top_mask

    xpk_ref[...] = lax.shift_right_logical(_rbf(u[:, :IN_DIM // 2]), 16) \
        | _rbf(u[:, IN_DIM // 2:])

    g = _gelu(jnp.dot(x, Wg1_ref[...], preferred_element_type=jnp.float32)
              + bg1_ref[...])
    gl = jnp.dot(g, Wg2_ref[...], preferred_element_type=jnp.float32) \
        + bg2_ref[...]

    iota_e = lax.broadcasted_iota(jnp.int32, gl.shape, 1)
    m1 = jnp.max(gl, axis=-1, keepdims=True)
    i1 = jnp.min(jnp.where(gl == m1, iota_e, E), axis=-1, keepdims=True)
    oh1 = (iota_e == i1)
    masked = jnp.where(oh1, -jnp.inf, gl)
    m2 = jnp.max(masked, axis=-1, keepdims=True)
    i2 = jnp.min(jnp.where(masked == m2, iota_e, E), axis=-1, keepdims=True)
    oh2 = (iota_e == i2)
    e2 = jnp.exp(m2 - m1)
    w1 = 1.0 / (1.0 + e2)
    w0_ref[...] = w1[:, 0]
    w1_ref[...] = (e2 * w1)[:, 0]
    iota_f = iota_e.astype(jnp.float32)
    e0_ref[...] = jnp.sum(jnp.where(oh1, iota_f, 0.0), axis=1)
    e1_ref[...] = jnp.sum(jnp.where(oh2, iota_f, 0.0), axis=1)

    osum = oh1.astype(jnp.float32) + oh2.astype(jnp.float32)
    inc = osum
    s = 1
    while s < _RB:
        inc = inc + jnp.concatenate(
            [jnp.zeros((s, E), jnp.float32), inc[:-s, :]], axis=0)
        s *= 2
    prev = jnp.where(tb == 0, jnp.zeros((1, E), jnp.float32), carry_ref[...])
    excl = (inc - osum) + prev
    sel0_ref[...] = jnp.sum(jnp.where(oh1, excl, 0.0), axis=1)
    sel1_ref[...] = jnp.sum(jnp.where(oh2, excl, 0.0), axis=1)
    new_carry = prev + inc[_RB - 1:_RB, :]
    carry_ref[...] = new_carry

    @pl.when(tb == _RTB - 1)
    def _():
        counts_ref[...] = new_carry


def _route_b_kernel(counts_ref, e0_ref, e1_ref, sel0_ref, sel1_ref,
                    p0_ref, p1_ref, be_ref):
    counts = counts_ref[...]
    padded = jnp.floor((counts + (BLK - 1)) / BLK) * BLK
    tri_e = (lax.broadcasted_iota(jnp.int32, (E, E), 0)
             < lax.broadcasted_iota(jnp.int32, (E, E), 1)).astype(jnp.float32)
    off = jnp.dot(padded, tri_e, preferred_element_type=jnp.float32)
    end = off + padded

    iota_e = lax.broadcasted_iota(jnp.int32, (TOKENS, E), 1).astype(jnp.float32)
    oh0 = (iota_e == e0_ref[...][:, None])
    oh1 = (iota_e == e1_ref[...][:, None])
    p0_ref[...] = (jnp.sum(jnp.where(oh0, off, 0.0), axis=1)
                   + sel0_ref[...]).astype(jnp.int32)
    p1_ref[...] = (jnp.sum(jnp.where(oh1, off, 0.0), axis=1)
                   + sel1_ref[...]).astype(jnp.int32)

    jb = lax.broadcasted_iota(jnp.int32, (NBLK, 1), 0).astype(jnp.float32) * BLK
    be = jnp.sum((end <= jb).astype(jnp.int32), axis=1)
    be_ref[...] = jnp.minimum(be, E - 1)


def _route(x, Wg1, bg1, Wg2, bg2):
    w0, w1, e0, e1, sel0, sel1, counts, xpk = pl.pallas_call(
        _route_a_kernel,
        grid=(_RTB,),
        in_specs=[
            pl.BlockSpec((_RB, IN_DIM), lambda tb: (tb, 0)),
            pl.BlockSpec((IN_DIM, GATE_H), lambda tb: (0, 0)),
            pl.BlockSpec((1, GATE_H), lambda tb: (0, 0)),
            pl.BlockSpec((GATE_H, E), lambda tb: (0, 0)),
            pl.BlockSpec((1, E), lambda tb: (0, 0)),
        ],
        out_specs=(
            pl.BlockSpec((_RB,), lambda tb: (tb,)),
            pl.BlockSpec((_RB,), lambda tb: (tb,)),
            pl.BlockSpec((_RB,), lambda tb: (tb,)),
            pl.BlockSpec((_RB,), lambda tb: (tb,)),
            pl.BlockSpec((_RB,), lambda tb: (tb,)),
            pl.BlockSpec((_RB,), lambda tb: (tb,)),
            pl.BlockSpec((1, E), lambda tb: (0, 0)),
            pl.BlockSpec((_RB, IN_DIM // 2), lambda tb: (tb, 0)),
        ),
        out_shape=(
            jax.ShapeDtypeStruct((TOKENS,), jnp.float32),
            jax.ShapeDtypeStruct((TOKENS,), jnp.float32),
            jax.ShapeDtypeStruct((TOKENS,), jnp.float32),
            jax.ShapeDtypeStruct((TOKENS,), jnp.float32),
            jax.ShapeDtypeStruct((TOKENS,), jnp.float32),
            jax.ShapeDtypeStruct((TOKENS,), jnp.float32),
            jax.ShapeDtypeStruct((1, E), jnp.float32),
            jax.ShapeDtypeStruct((TOKENS, IN_DIM // 2), jnp.int32),
        ),
        scratch_shapes=[pltpu.VMEM((1, E), jnp.float32)],
    )(x, Wg1, bg1.reshape(1, GATE_H), Wg2, bg2.reshape(1, E))
    p0, p1, be = pl.pallas_call(
        _route_b_kernel,
        out_shape=(
            jax.ShapeDtypeStruct((TOKENS,), jnp.int32),
            jax.ShapeDtypeStruct((TOKENS,), jnp.int32),
            jax.ShapeDtypeStruct((NBLK,), jnp.int32),
        ),
    )(counts, e0, e1, sel0, sel1)
    return w0, w1, p0, p1, be, xpk


_SC_MESH = dict(core_axis_name="c", subcore_axis_name="s")
_IN_P = IN_DIM // 2
_A_PER_W = ASSIGN // NW
_SCH = 32
_NSCH = _A_PER_W // _SCH


def _sc_wid():
    return lax.axis_index("s") * 2 + lax.axis_index("c")


def _k2_body(x_hbm, pos_hbm, xs_hbm, pv0, pv1, rb0, rb1, lsem, ssem):
    wid = _sc_wid()
    base = wid * _A_PER_W
    tok0 = pl.multiple_of(base & (TOKENS - 1), _SCH)
    pv = (pv0, pv1)
    rb = (rb0, rb1)
    lr = pltpu.async_copy(x_hbm.at[pl.ds(tok0, _SCH)], rb0, lsem)
    sh_prev = None
    for c in range(_NSCH):
        cur = c % 2
        nxt = (c + 1) % 2
        pltpu.sync_copy(pos_hbm.at[pl.ds(base + c * _SCH, _SCH)], pv[cur])
        lr.wait()
        sh = pltpu.async_copy(rb[cur], xs_hbm.at[pv[cur]], ssem)
        if c + 1 < _NSCH:
            if sh_prev is not None:
                sh_prev.wait()
            lr = pltpu.async_copy(
                x_hbm.at[pl.ds(tok0 + (c + 1) * _SCH, _SCH)], rb[nxt], lsem)
        else:
            if sh_prev is not None:
                sh_prev.wait()
        sh_prev = sh
    sh_prev.wait()


def _sc_dispatch(x_packed, pos_all):
    k = functools.partial(
        pl.kernel,
        mesh=plsc.VectorSubcoreMesh(**_SC_MESH),
        out_type=jax.ShapeDtypeStruct((CAP, _IN_P), jnp.int32),
        scratch_types=[
            pltpu.VMEM((_SCH,), jnp.int32),
            pltpu.VMEM((_SCH,), jnp.int32),
            pltpu.VMEM((_SCH, _IN_P), jnp.int32),
            pltpu.VMEM((_SCH, _IN_P), jnp.int32),
            pltpu.SemaphoreType.DMA,
            pltpu.SemaphoreType.DMA,
        ],
    )(_k2_body)
    return k(x_packed, pos_all)


def _mlp_kernel(be_ref, xs_ref, W1_ref, b1_ref, W2_ref, b2_ref,
                W3_ref, b3_ref, o30_ref, o31_ref):
    u = xs_ref[...]
    lo = lax.bitcast_convert_type(lax.shift_left(u, 16), jnp.float32)
    hi = lax.bitcast_convert_type(u & jnp.int32(-65536), jnp.float32)
    xs = jnp.concatenate([lo, hi], axis=1)
    h1 = _gelu(jnp.dot(xs, W1_ref[0],
                       preferred_element_type=jnp.float32) + b1_ref[0])
    h2 = _gelu(jnp.dot(h1, W2_ref[0],
                       preferred_element_type=jnp.float32) + b2_ref[0])
    w3 = W3_ref[0]
    b3 = b3_ref[0]
    o30_ref[...] = jnp.sum(h2 * w3[:, 0][None, :], axis=1) + b3[0, 0]
    o31_ref[...] = jnp.sum(h2 * w3[:, 1][None, :], axis=1) + b3[0, 1]


def _expert_mlp(be, xs, W1, b1, W2, b2, W3, b3):
    grid_spec = pltpu.PrefetchScalarGridSpec(
        num_scalar_prefetch=1,
        grid=(NBLK,),
        in_specs=[
            pl.BlockSpec((BLK, IN_DIM // 2), lambda j, be: (j, 0)),
            pl.BlockSpec((1, IN_DIM, HID), lambda j, be: (be[j], 0, 0)),
            pl.BlockSpec((1, 1, HID), lambda j, be: (be[j], 0, 0)),
            pl.BlockSpec((1, HID, HID // 2), lambda j, be: (be[j], 0, 0)),
            pl.BlockSpec((1, 1, HID // 2), lambda j, be: (be[j], 0, 0)),
            pl.BlockSpec((1, HID // 2, NC), lambda j, be: (be[j], 0, 0)),
            pl.BlockSpec((1, 1, NC), lambda j, be: (be[j], 0, 0)),
        ],
        out_specs=(
            pl.BlockSpec((BLK,), lambda j, be: (j,)),
            pl.BlockSpec((BLK,), lambda j, be: (j,)),
        ),
    )
    return pl.pallas_call(
        _mlp_kernel,
        grid_spec=grid_spec,
        out_shape=(
            jax.ShapeDtypeStruct((CAP,), jnp.float32),
            jax.ShapeDtypeStruct((CAP,), jnp.float32),
        ),
    )(be, xs, W1, b1.reshape(E, 1, HID), W2, b2.reshape(E, 1, HID // 2),
      W3, b3.reshape(E, 1, NC))


_TOK_PER_W = TOKENS // NW


def _k4_body(o30_hbm, o31_hbm, w0_hbm, w1_hbm, p0_hbm, p1_hbm, t_hbm,
             out_hbm, w0v, w1v, p0v, p1v, v00, v01, v10, v11, tv, ob, sem):
    wid = _sc_wid()
    tb = wid * _TOK_PER_W
    pltpu.sync_copy(w0_hbm.at[pl.ds(tb, _TOK_PER_W)], w0v)
    pltpu.sync_copy(w1_hbm.at[pl.ds(tb, _TOK_PER_W)], w1v)
    pltpu.sync_copy(p0_hbm.at[pl.ds(tb, _TOK_PER_W)], p0v)
    pltpu.sync_copy(p1_hbm.at[pl.ds(tb, _TOK_PER_W)], p1v)
    pltpu.sync_copy(t_hbm, tv)
    inv_t = 1.0 / jnp.maximum(tv[...], 1e-6)
    copies = []
    for pv, plane, dst in ((p0v, o30_hbm, v00), (p0v, o31_hbm, v01),
                           (p1v, o30_hbm, v10), (p1v, o31_hbm, v11)):
        copies.append(pltpu.async_copy(plane.at[pv], dst, sem))
    for cp in copies:
        cp.wait()
    for c, (a, b) in enumerate(((v00, v10), (v01, v11))):
        for g in range(_TOK_PER_W // 16):
            sl = pl.ds(g * 16, 16)
            ob[sl] = (w0v[sl] * a[sl] + w1v[sl] * b[sl]) * inv_t
        pltpu.sync_copy(ob, out_hbm.at[pl.ds(c * TOKENS + tb, _TOK_PER_W)])


def _sc_combine(o30, o31, w0, w1, p0, p1, temp16):
    k = functools.partial(
        pl.kernel,
        mesh=plsc.VectorSubcoreMesh(**_SC_MESH),
        out_type=jax.ShapeDtypeStruct((TOKENS * NC,), jnp.float32),
        scratch_types=[
            pltpu.VMEM((_TOK_PER_W,), jnp.float32),
            pltpu.VMEM((_TOK_PER_W,), jnp.float32),
            pltpu.VMEM((_TOK_PER_W,), jnp.int32),
            pltpu.VMEM((_TOK_PER_W,), jnp.int32),
            pltpu.VMEM((_TOK_PER_W,), jnp.float32),
            pltpu.VMEM((_TOK_PER_W,), jnp.float32),
            pltpu.VMEM((_TOK_PER_W,), jnp.float32),
            pltpu.VMEM((_TOK_PER_W,), jnp.float32),
            pltpu.VMEM((16,), jnp.float32),
            pltpu.VMEM((_TOK_PER_W,), jnp.float32),
            pltpu.SemaphoreType.DMA,
        ],
    )(_k4_body)
    return k(o30, o31, w0, w1, p0, p1, temp16)


def kernel(x, W1, b1, W2, b2, W3, b3, Wg1, bg1, Wg2, bg2, temperature):
    w0, w1, p0, p1, be, x_packed = _route(x, Wg1, bg1, Wg2, bg2)
    pos_all = jnp.concatenate([p0, p1])
    xs_packed = _sc_dispatch(x_packed, pos_all)
    o30, o31 = _expert_mlp(be, xs_packed, W1, b1, W2, b2, W3, b3)
    temp16 = jnp.broadcast_to(temperature.reshape(1), (16,))
    out = _sc_combine(o30, o31, w0, w1, p0, p1, temp16)
    return out.reshape(NC, TOKENS).T


# speedup vs baseline: 4.5722x; 1.0173x over previous
import functools

import jax
import jax.numpy as jnp
from jax import lax
from jax.experimental import pallas as pl
from jax.experimental.pallas import tpu as pltpu
from jax.experimental.pallas import tpu_sc as plsc

IN_DIM = 2048
HID = 1024
E = 8
NC = 2
GATE_H = 256
TOKENS = 4096
ASSIGN = 2 * TOKENS
BLK = 256
NBLK = ASSIGN // BLK + E
CAP = NBLK * BLK
PAD = CAP - ASSIGN
NW = 32


def _gelu(v):
    return v * 0.5 * (1.0 + lax.erf(v * 0.7071067811865476))


_RB = 512
_RTB = TOKENS // _RB


def _route_kernel(x_ref, Wg1_ref, bg1_ref, Wg2_ref, bg2_ref,
                  w0_ref, w1_ref, p0_ref, p1_ref, be_ref, xpk_ref,
                  carry_ref, e0_s, e1_s, sel0_s, sel1_s):
    tb = pl.program_id(0)

    @pl.when(tb < _RTB)
    def _main():
        x = x_ref[...]

        u = lax.bitcast_convert_type(x, jnp.int32)
        top_mask = jnp.int32(-65536)

        def _rbf(v):
            return (v + 0x7FFF
                    + (lax.shift_right_logical(v, 16) & 1)) & top_mask

        xpk_ref[...] = lax.shift_right_logical(_rbf(u[:, :IN_DIM // 2]), 16) \
            | _rbf(u[:, IN_DIM // 2:])

        g = _gelu(jnp.dot(x, Wg1_ref[...], preferred_element_type=jnp.float32)
                  + bg1_ref[...])
        gl = jnp.dot(g, Wg2_ref[...], preferred_element_type=jnp.float32) \
            + bg2_ref[...]

        iota_e = lax.broadcasted_iota(jnp.int32, gl.shape, 1)
        m1 = jnp.max(gl, axis=-1, keepdims=True)
        i1 = jnp.min(jnp.where(gl == m1, iota_e, E), axis=-1, keepdims=True)
        oh1 = (iota_e == i1)
        masked = jnp.where(oh1, -jnp.inf, gl)
        m2 = jnp.max(masked, axis=-1, keepdims=True)
        i2 = jnp.min(jnp.where(masked == m2, iota_e, E), axis=-1,
                     keepdims=True)
        oh2 = (iota_e == i2)
        e2 = jnp.exp(m2 - m1)
        w1 = 1.0 / (1.0 + e2)
        w0_ref[...] = w1[:, 0]
        w1_ref[...] = (e2 * w1)[:, 0]
        rows = pl.ds(tb * _RB, _RB)
        iota_f = iota_e.astype(jnp.float32)
        e0_s[rows] = jnp.sum(jnp.where(oh1, iota_f, 0.0), axis=1)
        e1_s[rows] = jnp.sum(jnp.where(oh2, iota_f, 0.0), axis=1)

        osum = oh1.astype(jnp.float32) + oh2.astype(jnp.float32)
        inc = osum
        s = 1
        while s < _RB:
            inc = inc + jnp.concatenate(
                [jnp.zeros((s, E), jnp.float32), inc[:-s, :]], axis=0)
            s *= 2
        prev = jnp.where(tb == 0, jnp.zeros((1, E), jnp.float32),
                         carry_ref[...])
        excl = (inc - osum) + prev
        sel0_s[rows] = jnp.sum(jnp.where(oh1, excl, 0.0), axis=1)
        sel1_s[rows] = jnp.sum(jnp.where(oh2, excl, 0.0), axis=1)
        carry_ref[...] = prev + inc[_RB - 1:_RB, :]

    @pl.when(tb == _RTB)
    def _finalize():
        counts = carry_ref[...]
        padded = jnp.floor((counts + (BLK - 1)) / BLK) * BLK
        tri_e = (lax.broadcasted_iota(jnp.int32, (E, E), 0)
                 < lax.broadcasted_iota(jnp.int32, (E, E), 1)
                 ).astype(jnp.float32)
        off = jnp.dot(padded, tri_e, preferred_element_type=jnp.float32)
        end = off + padded

        iota_e = lax.broadcasted_iota(jnp.int32, (TOKENS, E), 1) \
            .astype(jnp.float32)
        oh0 = (iota_e == e0_s[...][:, None])
        oh1 = (iota_e == e1_s[...][:, None])
        p0_ref[...] = (jnp.sum(jnp.where(oh0, off, 0.0), axis=1)
                       + sel0_s[...]).astype(jnp.int32)
        p1_ref[...] = (jnp.sum(jnp.where(oh1, off, 0.0), axis=1)
                       + sel1_s[...]).astype(jnp.int32)

        jb = lax.broadcasted_iota(jnp.int32, (NBLK + 1, 1), 0) \
            .astype(jnp.float32) * BLK
        be = jnp.sum((end <= jb[:NBLK]).astype(jnp.int32), axis=1)
        used = (jnp.sum(padded) / BLK).astype(jnp.int32)
        be_ref[...] = jnp.concatenate(
            [jnp.minimum(be, E - 1),
             jnp.broadcast_to(used[None], (1,))], axis=0)


def _route(x, Wg1, bg1, Wg2, bg2):
    _last = _RTB - 1
    return pl.pallas_call(
        _route_kernel,
        grid=(_RTB + 1,),
        in_specs=[
            pl.BlockSpec((_RB, IN_DIM), lambda tb: (jnp.minimum(tb, _last), 0)),
            pl.BlockSpec((IN_DIM, GATE_H), lambda tb: (0, 0)),
            pl.BlockSpec((1, GATE_H), lambda tb: (0, 0)),
            pl.BlockSpec((GATE_H, E), lambda tb: (0, 0)),
            pl.BlockSpec((1, E), lambda tb: (0, 0)),
        ],
        out_specs=(
            pl.BlockSpec((_RB,), lambda tb: (jnp.minimum(tb, _last),)),
            pl.BlockSpec((_RB,), lambda tb: (jnp.minimum(tb, _last),)),
            pl.BlockSpec((TOKENS,), lambda tb: (0,)),
            pl.BlockSpec((TOKENS,), lambda tb: (0,)),
            pl.BlockSpec((NBLK + 1,), lambda tb: (0,)),
            pl.BlockSpec((_RB, IN_DIM // 2),
                         lambda tb: (jnp.minimum(tb, _last), 0)),
        ),
        out_shape=(
            jax.ShapeDtypeStruct((TOKENS,), jnp.float32),
            jax.ShapeDtypeStruct((TOKENS,), jnp.float32),
            jax.ShapeDtypeStruct((TOKENS,), jnp.int32),
            jax.ShapeDtypeStruct((TOKENS,), jnp.int32),
            jax.ShapeDtypeStruct((NBLK + 1,), jnp.int32),
            jax.ShapeDtypeStruct((TOKENS, IN_DIM // 2), jnp.int32),
        ),
        scratch_shapes=[
            pltpu.VMEM((1, E), jnp.float32),
            pltpu.VMEM((TOKENS,), jnp.float32),
            pltpu.VMEM((TOKENS,), jnp.float32),
            pltpu.VMEM((TOKENS,), jnp.float32),
            pltpu.VMEM((TOKENS,), jnp.float32),
        ],
    )(x, Wg1, bg1.reshape(1, GATE_H), Wg2, bg2.reshape(1, E))


_SC_MESH = dict(core_axis_name="c", subcore_axis_name="s")
_IN_P = IN_DIM // 2
_A_PER_W = ASSIGN // NW
_SCH = 32
_NSCH = _A_PER_W // _SCH


def _sc_wid():
    return lax.axis_index("s") * 2 + lax.axis_index("c")


def _k2_body(x_hbm, pos_hbm, xs_hbm, pv0, pv1, rb0, rb1, lsem, ssem):
    wid = _sc_wid()
    base = wid * _A_PER_W
    tok0 = pl.multiple_of(base & (TOKENS - 1), _SCH)
    pv = (pv0, pv1)
    rb = (rb0, rb1)
    lr = pltpu.async_copy(x_hbm.at[pl.ds(tok0, _SCH)], rb0, lsem)
    sh_prev = None
    for c in range(_NSCH):
        cur = c % 2
        nxt = (c + 1) % 2
        pltpu.sync_copy(pos_hbm.at[pl.ds(base + c * _SCH, _SCH)], pv[cur])
        lr.wait()
        sh = pltpu.async_copy(rb[cur], xs_hbm.at[pv[cur]], ssem)
        if c + 1 < _NSCH:
            if sh_prev is not None:
                sh_prev.wait()
            lr = pltpu.async_copy(
                x_hbm.at[pl.ds(tok0 + (c + 1) * _SCH, _SCH)], rb[nxt], lsem)
        else:
            if sh_prev is not None:
                sh_prev.wait()
        sh_prev = sh
    sh_prev.wait()


def _sc_dispatch(x_packed, pos_all):
    k = functools.partial(
        pl.kernel,
        mesh=plsc.VectorSubcoreMesh(**_SC_MESH),
        out_type=jax.ShapeDtypeStruct((CAP, _IN_P), jnp.int32),
        scratch_types=[
            pltpu.VMEM((_SCH,), jnp.int32),
            pltpu.VMEM((_SCH,), jnp.int32),
            pltpu.VMEM((_SCH, _IN_P), jnp.int32),
            pltpu.VMEM((_SCH, _IN_P), jnp.int32),
            pltpu.SemaphoreType.DMA,
            pltpu.SemaphoreType.DMA,
        ],
    )(_k2_body)
    return k(x_packed, pos_all)


def _mlp_kernel(be_ref, xs_ref, W1_ref, b1_ref, W2_ref, b2_ref,
                W3_ref, b3_ref, o30_ref, o31_ref):
    @pl.when(pl.program_id(0) < be_ref[NBLK])
    def _():
        u = xs_ref[...]
        lo = lax.bitcast_convert_type(lax.shift_left(u, 16), jnp.float32)
        hi = lax.bitcast_convert_type(u & jnp.int32(-65536), jnp.float32)
        xs = jnp.concatenate([lo, hi], axis=1)
        h1 = _gelu(jnp.dot(xs, W1_ref[0],
                           preferred_element_type=jnp.float32) + b1_ref[0])
        h2 = _gelu(jnp.dot(h1, W2_ref[0],
                           preferred_element_type=jnp.float32) + b2_ref[0])
        w3 = W3_ref[0]
        b3 = b3_ref[0]
        o30_ref[...] = jnp.sum(h2 * w3[:, 0][None, :], axis=1) + b3[0, 0]
        o31_ref[...] = jnp.sum(h2 * w3[:, 1][None, :], axis=1) + b3[0, 1]


def _expert_mlp(be, xs, W1, b1, W2, b2, W3, b3):
    grid_spec = pltpu.PrefetchScalarGridSpec(
        num_scalar_prefetch=1,
        grid=(NBLK,),
        in_specs=[
            pl.BlockSpec((BLK, IN_DIM // 2), lambda j, be: (j, 0)),
            pl.BlockSpec((1, IN_DIM, HID), lambda j, be: (be[j], 0, 0)),
            pl.BlockSpec((1, 1, HID), lambda j, be: (be[j], 0, 0)),
            pl.BlockSpec((1, HID, HID // 2), lambda j, be: (be[j], 0, 0)),
            pl.BlockSpec((1, 1, HID // 2), lambda j, be: (be[j], 0, 0)),
            pl.BlockSpec((1, HID // 2, NC), lambda j, be: (be[j], 0, 0)),
            pl.BlockSpec((1, 1, NC), lambda j, be: (be[j], 0, 0)),
        ],
        out_specs=(
            pl.BlockSpec((BLK,), lambda j, be: (j,)),
            pl.BlockSpec((BLK,), lambda j, be: (j,)),
        ),
    )
    return pl.pallas_call(
        _mlp_kernel,
        grid_spec=grid_spec,
        out_shape=(
            jax.ShapeDtypeStruct((CAP,), jnp.float32),
            jax.ShapeDtypeStruct((CAP,), jnp.float32),
        ),
    )(be, xs, W1, b1.reshape(E, 1, HID), W2, b2.reshape(E, 1, HID // 2),
      W3, b3.reshape(E, 1, NC))


_TOK_PER_W = TOKENS // NW


def _k4_body(o30_hbm, o31_hbm, w0_hbm, w1_hbm, p0_hbm, p1_hbm, t_hbm,
             out_hbm, w0v, w1v, p0v, p1v, v00, v01, v10, v11, tv, ob, sem):
    wid = _sc_wid()
    tb = wid * _TOK_PER_W
    pltpu.sync_copy(w0_hbm.at[pl.ds(tb, _TOK_PER_W)], w0v)
    pltpu.sync_copy(w1_hbm.at[pl.ds(tb, _TOK_PER_W)], w1v)
    pltpu.sync_copy(p0_hbm.at[pl.ds(tb, _TOK_PER_W)], p0v)
    pltpu.sync_copy(p1_hbm.at[pl.ds(tb, _TOK_PER_W)], p1v)
    pltpu.sync_copy(t_hbm, tv)
    inv_t = 1.0 / jnp.maximum(tv[...], 1e-6)
    copies = []
    for pv, plane, dst in ((p0v, o30_hbm, v00), (p0v, o31_hbm, v01),
                           (p1v, o30_hbm, v10), (p1v, o31_hbm, v11)):
        copies.append(pltpu.async_copy(plane.at[pv], dst, sem))
    for cp in copies:
        cp.wait()
    for c, (a, b) in enumerate(((v00, v10), (v01, v11))):
        for g in range(_TOK_PER_W // 16):
            sl = pl.ds(g * 16, 16)
            ob[sl] = (w0v[sl] * a[sl] + w1v[sl] * b[sl]) * inv_t
        pltpu.sync_copy(ob, out_hbm.at[pl.ds(c * TOKENS + tb, _TOK_PER_W)])


def _sc_combine(o30, o31, w0, w1, p0, p1, temp16):
    k = functools.partial(
        pl.kernel,
        mesh=plsc.VectorSubcoreMesh(**_SC_MESH),
        out_type=jax.ShapeDtypeStruct((TOKENS * NC,), jnp.float32),
        scratch_types=[
            pltpu.VMEM((_TOK_PER_W,), jnp.float32),
            pltpu.VMEM((_TOK_PER_W,), jnp.float32),
            pltpu.VMEM((_TOK_PER_W,), jnp.int32),
            pltpu.VMEM((_TOK_PER_W,), jnp.int32),
            pltpu.VMEM((_TOK_PER_W,), jnp.float32),
            pltpu.VMEM((_TOK_PER_W,), jnp.float32),
            pltpu.VMEM((_TOK_PER_W,), jnp.float32),
            pltpu.VMEM((_TOK_PER_W,), jnp.float32),
            pltpu.VMEM((16,), jnp.float32),
            pltpu.VMEM((_TOK_PER_W,), jnp.float32),
            pltpu.SemaphoreType.DMA,
        ],
    )(_k4_body)
    return k(o30, o31, w0, w1, p0, p1, temp16)


def kernel(x, W1, b1, W2, b2, W3, b3, Wg1, bg1, Wg2, bg2, temperature):
    w0, w1, p0, p1, be_ext, x_packed = _route(x, Wg1, bg1, Wg2, bg2)
    pos_all = jnp.concatenate([p0, p1])
    xs_packed = _sc_dispatch(x_packed, pos_all)
    o30, o31 = _expert_mlp(be_ext, xs_packed, W1, b1, W2, b2, W3, b3)
    temp16 = jnp.broadcast_to(temperature.reshape(1), (16,))
    out = _sc_combine(o30, o31, w0, w1, p0, p1, temp16)
    return out.reshape(NC, TOKENS).T


# speedup vs baseline: 4.7556x; 1.0401x over previous
import functools

import jax
import jax.numpy as jnp
from jax import lax
from jax.experimental import pallas as pl
from jax.experimental.pallas import tpu as pltpu
from jax.experimental.pallas import tpu_sc as plsc

IN_DIM = 2048
HID = 1024
E = 8
NC = 2
GATE_H = 256
TOKENS = 4096
ASSIGN = 2 * TOKENS
BLK = 512
NBLK = ASSIGN // BLK + E
CAP = NBLK * BLK
PAD = CAP - ASSIGN
NW = 32


def _gelu(v):
    return v * 0.5 * (1.0 + lax.erf(v * 0.7071067811865476))


_RB = 512
_RTB = TOKENS // _RB


def _route_kernel(x_ref, Wg1_ref, bg1_ref, Wg2_ref, bg2_ref,
                  w0_ref, w1_ref, p0_ref, p1_ref, be_ref, xpk_ref,
                  carry_ref, e0_s, e1_s, sel0_s, sel1_s):
    tb = pl.program_id(0)

    @pl.when(tb < _RTB)
    def _main():
        x = x_ref[...]

        u = lax.bitcast_convert_type(x, jnp.int32)
        top_mask = jnp.int32(-65536)

        def _rbf(v):
            return (v + 0x7FFF
                    + (lax.shift_right_logical(v, 16) & 1)) & top_mask

        xpk_ref[...] = lax.shift_right_logical(_rbf(u[:, :IN_DIM // 2]), 16) \
            | _rbf(u[:, IN_DIM // 2:])

        g = _gelu(jnp.dot(x, Wg1_ref[...], preferred_element_type=jnp.float32)
                  + bg1_ref[...])
        gl = jnp.dot(g, Wg2_ref[...], preferred_element_type=jnp.float32) \
            + bg2_ref[...]

        iota_e = lax.broadcasted_iota(jnp.int32, gl.shape, 1)
        m1 = jnp.max(gl, axis=-1, keepdims=True)
        i1 = jnp.min(jnp.where(gl == m1, iota_e, E), axis=-1, keepdims=True)
        oh1 = (iota_e == i1)
        masked = jnp.where(oh1, -jnp.inf, gl)
        m2 = jnp.max(masked, axis=-1, keepdims=True)
        i2 = jnp.min(jnp.where(masked == m2, iota_e, E), axis=-1,
                     keepdims=True)
        oh2 = (iota_e == i2)
        e2 = jnp.exp(m2 - m1)
        w1 = 1.0 / (1.0 + e2)
        w0_ref[...] = w1[:, 0]
        w1_ref[...] = (e2 * w1)[:, 0]
        rows = pl.ds(tb * _RB, _RB)
        iota_f = iota_e.astype(jnp.float32)
        e0_s[rows] = jnp.sum(jnp.where(oh1, iota_f, 0.0), axis=1)
        e1_s[rows] = jnp.sum(jnp.where(oh2, iota_f, 0.0), axis=1)

        osum = oh1.astype(jnp.float32) + oh2.astype(jnp.float32)
        inc = osum
        s = 1
        while s < _RB:
            inc = inc + jnp.concatenate(
                [jnp.zeros((s, E), jnp.float32), inc[:-s, :]], axis=0)
            s *= 2
        prev = jnp.where(tb == 0, jnp.zeros((1, E), jnp.float32),
                         carry_ref[...])
        excl = (inc - osum) + prev
        sel0_s[rows] = jnp.sum(jnp.where(oh1, excl, 0.0), axis=1)
        sel1_s[rows] = jnp.sum(jnp.where(oh2, excl, 0.0), axis=1)
        carry_ref[...] = prev + inc[_RB - 1:_RB, :]

    @pl.when(tb == _RTB)
    def _finalize():
        counts = carry_ref[...]
        padded = jnp.floor((counts + (BLK - 1)) / BLK) * BLK
        tri_e = (lax.broadcasted_iota(jnp.int32, (E, E), 0)
                 < lax.broadcasted_iota(jnp.int32, (E, E), 1)
                 ).astype(jnp.float32)
        off = jnp.dot(padded, tri_e, preferred_element_type=jnp.float32)
        end = off + padded

        iota_e = lax.broadcasted_iota(jnp.int32, (TOKENS, E), 1) \
            .astype(jnp.float32)
        oh0 = (iota_e == e0_s[...][:, None])
        oh1 = (iota_e == e1_s[...][:, None])
        p0_ref[...] = (jnp.sum(jnp.where(oh0, off, 0.0), axis=1)
                       + sel0_s[...]).astype(jnp.int32)
        p1_ref[...] = (jnp.sum(jnp.where(oh1, off, 0.0), axis=1)
                       + sel1_s[...]).astype(jnp.int32)

        jb = lax.broadcasted_iota(jnp.int32, (NBLK + 1, 1), 0) \
            .astype(jnp.float32) * BLK
        be = jnp.sum((end <= jb[:NBLK]).astype(jnp.int32), axis=1)
        used = (jnp.sum(padded) / BLK).astype(jnp.int32)
        be_ref[...] = jnp.concatenate(
            [jnp.minimum(be, E - 1),
             jnp.broadcast_to(used[None], (1,))], axis=0)


def _route(x, Wg1, bg1, Wg2, bg2):
    _last = _RTB - 1
    return pl.pallas_call(
        _route_kernel,
        grid=(_RTB + 1,),
        in_specs=[
            pl.BlockSpec((_RB, IN_DIM), lambda tb: (jnp.minimum(tb, _last), 0)),
            pl.BlockSpec((IN_DIM, GATE_H), lambda tb: (0, 0)),
            pl.BlockSpec((1, GATE_H), lambda tb: (0, 0)),
            pl.BlockSpec((GATE_H, E), lambda tb: (0, 0)),
            pl.BlockSpec((1, E), lambda tb: (0, 0)),
        ],
        out_specs=(
            pl.BlockSpec((_RB,), lambda tb: (jnp.minimum(tb, _last),)),
            pl.BlockSpec((_RB,), lambda tb: (jnp.minimum(tb, _last),)),
            pl.BlockSpec((TOKENS,), lambda tb: (0,)),
            pl.BlockSpec((TOKENS,), lambda tb: (0,)),
            pl.BlockSpec((NBLK + 1,), lambda tb: (0,)),
            pl.BlockSpec((_RB, IN_DIM // 2),
                         lambda tb: (jnp.minimum(tb, _last), 0)),
        ),
        out_shape=(
            jax.ShapeDtypeStruct((TOKENS,), jnp.float32),
            jax.ShapeDtypeStruct((TOKENS,), jnp.float32),
            jax.ShapeDtypeStruct((TOKENS,), jnp.int32),
            jax.ShapeDtypeStruct((TOKENS,), jnp.int32),
            jax.ShapeDtypeStruct((NBLK + 1,), jnp.int32),
            jax.ShapeDtypeStruct((TOKENS, IN_DIM // 2), jnp.int32),
        ),
        scratch_shapes=[
            pltpu.VMEM((1, E), jnp.float32),
            pltpu.VMEM((TOKENS,), jnp.float32),
            pltpu.VMEM((TOKENS,), jnp.float32),
            pltpu.VMEM((TOKENS,), jnp.float32),
            pltpu.VMEM((TOKENS,), jnp.float32),
        ],
    )(x, Wg1, bg1.reshape(1, GATE_H), Wg2, bg2.reshape(1, E))


_SC_MESH = dict(core_axis_name="c", subcore_axis_name="s")
_IN_P = IN_DIM // 2
_A_PER_W = ASSIGN // NW
_SCH = 32
_NSCH = _A_PER_W // _SCH


def _sc_wid():
    return lax.axis_index("s") * 2 + lax.axis_index("c")


def _k2_body(x_hbm, pos_hbm, xs_hbm, pv0, pv1, rb0, rb1, lsem, ssem):
    wid = _sc_wid()
    base = wid * _A_PER_W
    tok0 = pl.multiple_of(base & (TOKENS - 1), _SCH)
    pv = (pv0, pv1)
    rb = (rb0, rb1)
    lr = pltpu.async_copy(x_hbm.at[pl.ds(tok0, _SCH)], rb0, lsem)
    sh_prev = None
    for c in range(_NSCH):
        cur = c % 2
        nxt = (c + 1) % 2
        pltpu.sync_copy(pos_hbm.at[pl.ds(base + c * _SCH, _SCH)], pv[cur])
        lr.wait()
        sh = pltpu.async_copy(rb[cur], xs_hbm.at[pv[cur]], ssem)
        if c + 1 < _NSCH:
            if sh_prev is not None:
                sh_prev.wait()
            lr = pltpu.async_copy(
                x_hbm.at[pl.ds(tok0 + (c + 1) * _SCH, _SCH)], rb[nxt], lsem)
        else:
            if sh_prev is not None:
                sh_prev.wait()
        sh_prev = sh
    sh_prev.wait()


def _sc_dispatch(x_packed, pos_all):
    k = functools.partial(
        pl.kernel,
        mesh=plsc.VectorSubcoreMesh(**_SC_MESH),
        out_type=jax.ShapeDtypeStruct((CAP, _IN_P), jnp.int32),
        scratch_types=[
            pltpu.VMEM((_SCH,), jnp.int32),
            pltpu.VMEM((_SCH,), jnp.int32),
            pltpu.VMEM((_SCH, _IN_P), jnp.int32),
            pltpu.VMEM((_SCH, _IN_P), jnp.int32),
            pltpu.SemaphoreType.DMA,
            pltpu.SemaphoreType.DMA,
        ],
    )(_k2_body)
    return k(x_packed, pos_all)


def _mlp_kernel(be_ref, xs_ref, W1_ref, b1_ref, W2_ref, b2_ref,
                W3_ref, b3_ref, o30_ref, o31_ref):
    @pl.when(pl.program_id(0) < be_ref[NBLK])
    def _():
        u = xs_ref[...]
        lo = lax.bitcast_convert_type(lax.shift_left(u, 16), jnp.float32)
        hi = lax.bitcast_convert_type(u & jnp.int32(-65536), jnp.float32)
        xs = jnp.concatenate([lo, hi], axis=1)
        h1 = _gelu(jnp.dot(xs, W1_ref[0],
                           preferred_element_type=jnp.float32) + b1_ref[0])
        h2 = _gelu(jnp.dot(h1, W2_ref[0],
                           preferred_element_type=jnp.float32) + b2_ref[0])
        w3 = W3_ref[0]
        b3 = b3_ref[0]
        o30_ref[...] = jnp.sum(h2 * w3[:, 0][None, :], axis=1) + b3[0, 0]
        o31_ref[...] = jnp.sum(h2 * w3[:, 1][None, :], axis=1) + b3[0, 1]


def _expert_mlp(be, xs, W1, b1, W2, b2, W3, b3):
    grid_spec = pltpu.PrefetchScalarGridSpec(
        num_scalar_prefetch=1,
        grid=(NBLK,),
        in_specs=[
            pl.BlockSpec((BLK, IN_DIM // 2), lambda j, be: (j, 0)),
            pl.BlockSpec((1, IN_DIM, HID), lambda j, be: (be[j], 0, 0)),
            pl.BlockSpec((1, 1, HID), lambda j, be: (be[j], 0, 0)),
            pl.BlockSpec((1, HID, HID // 2), lambda j, be: (be[j], 0, 0)),
            pl.BlockSpec((1, 1, HID // 2), lambda j, be: (be[j], 0, 0)),
            pl.BlockSpec((1, HID // 2, NC), lambda j, be: (be[j], 0, 0)),
            pl.BlockSpec((1, 1, NC), lambda j, be: (be[j], 0, 0)),
        ],
        out_specs=(
            pl.BlockSpec((BLK,), lambda j, be: (j,)),
            pl.BlockSpec((BLK,), lambda j, be: (j,)),
        ),
    )
    return pl.pallas_call(
        _mlp_kernel,
        grid_spec=grid_spec,
        out_shape=(
            jax.ShapeDtypeStruct((CAP,), jnp.float32),
            jax.ShapeDtypeStruct((CAP,), jnp.float32),
        ),
    )(be, xs, W1, b1.reshape(E, 1, HID), W2, b2.reshape(E, 1, HID // 2),
      W3, b3.reshape(E, 1, NC))


_TOK_PER_W = TOKENS // NW


def _k4_body(o30_hbm, o31_hbm, w0_hbm, w1_hbm, p0_hbm, p1_hbm, t_hbm,
             out_hbm, w0v, w1v, p0v, p1v, v00, v01, v10, v11, tv, ob, sem):
    wid = _sc_wid()
    tb = wid * _TOK_PER_W
    pltpu.sync_copy(w0_hbm.at[pl.ds(tb, _TOK_PER_W)], w0v)
    pltpu.sync_copy(w1_hbm.at[pl.ds(tb, _TOK_PER_W)], w1v)
    pltpu.sync_copy(p0_hbm.at[pl.ds(tb, _TOK_PER_W)], p0v)
    pltpu.sync_copy(p1_hbm.at[pl.ds(tb, _TOK_PER_W)], p1v)
    pltpu.sync_copy(t_hbm, tv)
    inv_t = 1.0 / jnp.maximum(tv[...], 1e-6)
    copies = []
    for pv, plane, dst in ((p0v, o30_hbm, v00), (p0v, o31_hbm, v01),
                           (p1v, o30_hbm, v10), (p1v, o31_hbm, v11)):
        copies.append(pltpu.async_copy(plane.at[pv], dst, sem))
    for cp in copies:
        cp.wait()
    for c, (a, b) in enumerate(((v00, v10), (v01, v11))):
        for g in range(_TOK_PER_W // 16):
            sl = pl.ds(g * 16, 16)
            ob[sl] = (w0v[sl] * a[sl] + w1v[sl] * b[sl]) * inv_t
        pltpu.sync_copy(ob, out_hbm.at[pl.ds(c * TOKENS + tb, _TOK_PER_W)])


def _sc_combine(o30, o31, w0, w1, p0, p1, temp16):
    k = functools.partial(
        pl.kernel,
        mesh=plsc.VectorSubcoreMesh(**_SC_MESH),
        out_type=jax.ShapeDtypeStruct((TOKENS * NC,), jnp.float32),
        scratch_types=[
            pltpu.VMEM((_TOK_PER_W,), jnp.float32),
            pltpu.VMEM((_TOK_PER_W,), jnp.float32),
            pltpu.VMEM((_TOK_PER_W,), jnp.int32),
            pltpu.VMEM((_TOK_PER_W,), jnp.int32),
            pltpu.VMEM((_TOK_PER_W,), jnp.float32),
            pltpu.VMEM((_TOK_PER_W,), jnp.float32),
            pltpu.VMEM((_TOK_PER_W,), jnp.float32),
            pltpu.VMEM((_TOK_PER_W,), jnp.float32),
            pltpu.VMEM((16,), jnp.float32),
            pltpu.VMEM((_TOK_PER_W,), jnp.float32),
            pltpu.SemaphoreType.DMA,
        ],
    )(_k4_body)
    return k(o30, o31, w0, w1, p0, p1, temp16)


def kernel(x, W1, b1, W2, b2, W3, b3, Wg1, bg1, Wg2, bg2, temperature):
    w0, w1, p0, p1, be_ext, x_packed = _route(x, Wg1, bg1, Wg2, bg2)
    pos_all = jnp.concatenate([p0, p1])
    xs_packed = _sc_dispatch(x_packed, pos_all)
    o30, o31 = _expert_mlp(be_ext, xs_packed, W1, b1, W2, b2, W3, b3)
    temp16 = jnp.broadcast_to(temperature.reshape(1), (16,))
    out = _sc_combine(o30, o31, w0, w1, p0, p1, temp16)
    return out.reshape(NC, TOKENS).T
